# baseline (speedup 1.0000x reference)
import jax
import jax.numpy as jnp
from jax import lax
from jax.experimental import pallas as pl
from jax.experimental.pallas import tpu as pltpu

B = 64
N = 1_000_000
K = 5
BLK = 16384
NB = -(-N // BLK)
GBLK = 512


def _gather_body(tgt_ref, x_ref, tv_ref):
    b = pl.program_id(0)
    off = tgt_ref[b] % GBLK
    x = x_ref[...]
    riota = lax.broadcasted_iota(jnp.int32, (8, GBLK), 0)
    ciota = lax.broadcasted_iota(jnp.int32, (8, GBLK), 1)
    mask = (riota == b % 8) & (ciota == off)
    val = jnp.max(jnp.where(mask, x, -jnp.inf))
    out_iota = lax.broadcasted_iota(jnp.int32, (B, 1), 0)
    tv_ref[...] = jnp.where(out_iota == b, val, tv_ref[...])


def _count_fast_body(tv_ref, ones_ref, x_ref, acc_ref, flag_ref,
                     sgt_ref, sge_ref):
    j = pl.program_id(0)

    @pl.when(j == 0)
    def _init():
        sgt_ref[...] = jnp.zeros_like(sgt_ref)
        sge_ref[...] = jnp.zeros_like(sge_ref)

    x = x_ref[...]
    tv = tv_ref[...]
    ones = ones_ref[...]

    @pl.when(j < NB - 1)
    def _mid():
        gt = (x > tv).astype(jnp.float32)
        ge = (x >= tv).astype(jnp.float32)
        sgt_ref[...] += jnp.dot(gt, ones, preferred_element_type=jnp.float32)
        sge_ref[...] += jnp.dot(ge, ones, preferred_element_type=jnp.float32)

    @pl.when(j == NB - 1)
    def _last():
        iota = lax.broadcasted_iota(jnp.int32, (B, BLK), 1)
        valid = iota < (N - j * BLK)
        gt = ((x > tv) & valid).astype(jnp.float32)
        ge = ((x >= tv) & valid).astype(jnp.float32)
        cnt_gt = sgt_ref[...] + jnp.dot(gt, ones, preferred_element_type=jnp.float32)
        cnt_ge = sge_ref[...] + jnp.dot(ge, ones, preferred_element_type=jnp.float32)
        cnt_eq = cnt_ge - cnt_gt
        hits = (cnt_gt < K).astype(jnp.float32)
        acc_ref[...] = (jnp.sum(hits) * (1.0 / B)).reshape(1, 1)
        risky = (cnt_gt < K) & (cnt_gt + cnt_eq - 1.0 >= K)
        flag_ref[...] = jnp.max(risky.astype(jnp.float32)).reshape(1, 1)


def _count_exact_body(tv_ref, tvm_ref, tgt_ref, x_ref, out_ref, acc_ref):
    j = pl.program_id(0)

    @pl.when(j == 0)
    def _init():
        acc_ref[...] = jnp.zeros_like(acc_ref)

    x = x_ref[...]
    tv = tv_ref[...]
    tvm = tvm_ref[...]
    tb = tgt_ref[...] - j * BLK
    iota = lax.broadcasted_iota(jnp.int32, (B, BLK), 1)
    thr = jnp.where(iota < tb, tvm, tv)

    @pl.when(j < NB - 1)
    def _mid():
        beats = (x > thr).astype(jnp.float32)
        acc_ref[...] += jnp.sum(beats, axis=1, keepdims=True)

    @pl.when(j == NB - 1)
    def _last():
        thr2 = jnp.where(iota < (N - j * BLK), thr, jnp.inf)
        beats = (x > thr2).astype(jnp.float32)
        counts = acc_ref[...] + jnp.sum(beats, axis=1, keepdims=True)
        hits = (counts < K).astype(jnp.float32)
        out_ref[...] = (jnp.sum(hits) * (1.0 / B)).reshape(1, 1)


def kernel(logits, target):
    tgt = target.astype(jnp.int32)

    grid_spec = pltpu.PrefetchScalarGridSpec(
        num_scalar_prefetch=1,
        grid=(B,),
        in_specs=[pl.BlockSpec((8, GBLK), lambda b, t: (b // 8, t[b] // GBLK))],
        out_specs=pl.BlockSpec((B, 1), lambda b, t: (0, 0)),
    )
    tv = pl.pallas_call(
        _gather_body,
        grid_spec=grid_spec,
        out_shape=jax.ShapeDtypeStruct((B, 1), jnp.float32),
    )(tgt, logits)

    ones = jnp.ones((BLK, 1), jnp.float32)

    acc_fast, flag = pl.pallas_call(
        _count_fast_body,
        grid=(NB,),
        in_specs=[
            pl.BlockSpec((B, 1), lambda j: (0, 0)),
            pl.BlockSpec((BLK, 1), lambda j: (0, 0)),
            pl.BlockSpec((B, BLK), lambda j: (0, j)),
        ],
        out_specs=[
            pl.BlockSpec((1, 1), lambda j: (0, 0)),
            pl.BlockSpec((1, 1), lambda j: (0, 0)),
        ],
        out_shape=[
            jax.ShapeDtypeStruct((1, 1), jnp.float32),
            jax.ShapeDtypeStruct((1, 1), jnp.float32),
        ],
        scratch_shapes=[
            pltpu.VMEM((B, 1), jnp.float32),
            pltpu.VMEM((B, 1), jnp.float32),
        ],
    )(tv, ones, logits)

    def _exact(_):
        tvm = jnp.nextafter(tv, jnp.float32(-jnp.inf))
        acc = pl.pallas_call(
            _count_exact_body,
            grid=(NB,),
            in_specs=[
                pl.BlockSpec((B, 1), lambda j: (0, 0)),
                pl.BlockSpec((B, 1), lambda j: (0, 0)),
                pl.BlockSpec((B, 1), lambda j: (0, 0)),
                pl.BlockSpec((B, BLK), lambda j: (0, j)),
            ],
            out_specs=pl.BlockSpec((1, 1), lambda j: (0, 0)),
            out_shape=jax.ShapeDtypeStruct((1, 1), jnp.float32),
            scratch_shapes=[pltpu.VMEM((B, 1), jnp.float32)],
        )(tv, tvm, tgt.reshape(B, 1), logits)
        return acc[0, 0]

    return lax.cond(flag[0, 0] > 0, _exact, lambda _: acc_fast[0, 0], None)


# speedup vs baseline: 1.2422x; 1.2422x over previous
import jax
import jax.numpy as jnp
from jax import lax
from jax.experimental import pallas as pl
from jax.experimental.pallas import tpu as pltpu

B = 64
N = 1_000_000
K = 5
BLK = 16384
NB = -(-N // BLK)
GBLK = 512


def _gather_body(tgt_ref, x_ref, tv_ref):
    b = pl.program_id(0)
    off = tgt_ref[b] % GBLK
    x = x_ref[...]
    riota = lax.broadcasted_iota(jnp.int32, (8, GBLK), 0)
    ciota = lax.broadcasted_iota(jnp.int32, (8, GBLK), 1)
    mask = (riota == b % 8) & (ciota == off)
    val = jnp.max(jnp.where(mask, x, -jnp.inf))
    out_iota = lax.broadcasted_iota(jnp.int32, (B, 1), 0)
    tv_ref[...] = jnp.where(out_iota == b, val, tv_ref[...])


def _count_fast_body(tv_ref, x_ref, acc_ref, flag_ref, sgt_ref, sge_ref):
    j = pl.program_id(0)

    @pl.when(j == 0)
    def _init():
        sgt_ref[...] = jnp.zeros_like(sgt_ref)
        sge_ref[...] = jnp.zeros_like(sge_ref)

    x = x_ref[...]
    tv = tv_ref[...]

    @pl.when(j < NB - 1)
    def _mid():
        gt = (x > tv).astype(jnp.float32)
        ge = (x >= tv).astype(jnp.float32)
        sgt_ref[...] += jnp.sum(gt, axis=1, keepdims=True)
        sge_ref[...] += jnp.sum(ge, axis=1, keepdims=True)

    @pl.when(j == NB - 1)
    def _last():
        iota = lax.broadcasted_iota(jnp.int32, (B, BLK), 1)
        valid = iota < (N - j * BLK)
        gt = ((x > tv) & valid).astype(jnp.float32)
        ge = ((x >= tv) & valid).astype(jnp.float32)
        cnt_gt = sgt_ref[...] + jnp.sum(gt, axis=1, keepdims=True)
        cnt_ge = sge_ref[...] + jnp.sum(ge, axis=1, keepdims=True)
        cnt_eq = cnt_ge - cnt_gt
        hits = (cnt_gt < K).astype(jnp.float32)
        acc_ref[...] = (jnp.sum(hits) * (1.0 / B)).reshape(1, 1)
        risky = (cnt_gt < K) & (cnt_gt + cnt_eq - 1.0 >= K)
        flag_ref[...] = jnp.max(risky.astype(jnp.float32)).reshape(1, 1)


def _count_exact_body(tv_ref, tvm_ref, tgt_ref, x_ref, out_ref, acc_ref):
    j = pl.program_id(0)

    @pl.when(j == 0)
    def _init():
        acc_ref[...] = jnp.zeros_like(acc_ref)

    x = x_ref[...]
    tv = tv_ref[...]
    tvm = tvm_ref[...]
    tb = tgt_ref[...] - j * BLK
    iota = lax.broadcasted_iota(jnp.int32, (B, BLK), 1)
    thr = jnp.where(iota < tb, tvm, tv)

    @pl.when(j < NB - 1)
    def _mid():
        beats = (x > thr).astype(jnp.float32)
        acc_ref[...] += jnp.sum(beats, axis=1, keepdims=True)

    @pl.when(j == NB - 1)
    def _last():
        thr2 = jnp.where(iota < (N - j * BLK), thr, jnp.inf)
        beats = (x > thr2).astype(jnp.float32)
        counts = acc_ref[...] + jnp.sum(beats, axis=1, keepdims=True)
        hits = (counts < K).astype(jnp.float32)
        out_ref[...] = (jnp.sum(hits) * (1.0 / B)).reshape(1, 1)


def kernel(logits, target):
    tgt = target.astype(jnp.int32)

    grid_spec = pltpu.PrefetchScalarGridSpec(
        num_scalar_prefetch=1,
        grid=(B,),
        in_specs=[pl.BlockSpec((8, GBLK), lambda b, t: (b // 8, t[b] // GBLK))],
        out_specs=pl.BlockSpec((B, 1), lambda b, t: (0, 0)),
    )
    tv = pl.pallas_call(
        _gather_body,
        grid_spec=grid_spec,
        out_shape=jax.ShapeDtypeStruct((B, 1), jnp.float32),
    )(tgt, logits)

    acc_fast, flag = pl.pallas_call(
        _count_fast_body,
        grid=(NB,),
        in_specs=[
            pl.BlockSpec((B, 1), lambda j: (0, 0)),
            pl.BlockSpec((B, BLK), lambda j: (0, j)),
        ],
        out_specs=[
            pl.BlockSpec((1, 1), lambda j: (0, 0)),
            pl.BlockSpec((1, 1), lambda j: (0, 0)),
        ],
        out_shape=[
            jax.ShapeDtypeStruct((1, 1), jnp.float32),
            jax.ShapeDtypeStruct((1, 1), jnp.float32),
        ],
        scratch_shapes=[
            pltpu.VMEM((B, 1), jnp.float32),
            pltpu.VMEM((B, 1), jnp.float32),
        ],
    )(tv, logits)

    def _exact(_):
        tvm = jnp.nextafter(tv, jnp.float32(-jnp.inf))
        acc = pl.pallas_call(
            _count_exact_body,
            grid=(NB,),
            in_specs=[
                pl.BlockSpec((B, 1), lambda j: (0, 0)),
                pl.BlockSpec((B, 1), lambda j: (0, 0)),
                pl.BlockSpec((B, 1), lambda j: (0, 0)),
                pl.BlockSpec((B, BLK), lambda j: (0, j)),
            ],
            out_specs=pl.BlockSpec((1, 1), lambda j: (0, 0)),
            out_shape=jax.ShapeDtypeStruct((1, 1), jnp.float32),
            scratch_shapes=[pltpu.VMEM((B, 1), jnp.float32)],
        )(tv, tvm, tgt.reshape(B, 1), logits)
        return acc[0, 0]

    return lax.cond(flag[0, 0] > 0, _exact, lambda _: acc_fast[0, 0], None)


# speedup vs baseline: 1.3602x; 1.0950x over previous
import jax
import jax.numpy as jnp
from jax import lax
from jax.experimental import pallas as pl
from jax.experimental.pallas import tpu as pltpu

B = 64
N = 1_000_000
K = 5
BLK = 16384
NB = -(-N // BLK)
GBLK = 512


def _gather_body(tgt_ref, x_ref, tv_ref):
    b = pl.program_id(0)
    off = tgt_ref[b] % GBLK
    x = x_ref[...]
    riota = lax.broadcasted_iota(jnp.int32, (8, GBLK), 0)
    ciota = lax.broadcasted_iota(jnp.int32, (8, GBLK), 1)
    mask = (riota == b % 8) & (ciota == off)
    val = jnp.max(jnp.where(mask, x, -jnp.inf))
    out_iota = lax.broadcasted_iota(jnp.int32, (B, 1), 0)
    tv_ref[...] = jnp.where(out_iota == b, val, tv_ref[...])


def _count_fast_body(tv_ref, x_ref, flag_ref, sgt_ref):
    j = pl.program_id(0)

    @pl.when(j == 0)
    def _init():
        sgt_ref[...] = jnp.zeros_like(sgt_ref)

    x = x_ref[...]
    tv = tv_ref[...]

    @pl.when(j < NB - 1)
    def _mid():
        gt = (x > tv).astype(jnp.float32)
        sgt_ref[...] += jnp.sum(gt, axis=1, keepdims=True)

    @pl.when(j == NB - 1)
    def _last():
        iota = lax.broadcasted_iota(jnp.int32, (B, BLK), 1)
        valid = iota < (N - j * BLK)
        gt = ((x > tv) & valid).astype(jnp.float32)
        cnt_gt = sgt_ref[...] + jnp.sum(gt, axis=1, keepdims=True)
        maybe_hit = (cnt_gt < K).astype(jnp.float32)
        flag_ref[...] = jnp.max(maybe_hit).reshape(1, 1)


def _count_exact_body(tv_ref, tvm_ref, tgt_ref, x_ref, out_ref, acc_ref):
    j = pl.program_id(0)

    @pl.when(j == 0)
    def _init():
        acc_ref[...] = jnp.zeros_like(acc_ref)

    x = x_ref[...]
    tv = tv_ref[...]
    tvm = tvm_ref[...]
    tb = tgt_ref[...] - j * BLK
    iota = lax.broadcasted_iota(jnp.int32, (B, BLK), 1)
    thr = jnp.where(iota < tb, tvm, tv)

    @pl.when(j < NB - 1)
    def _mid():
        beats = (x > thr).astype(jnp.float32)
        acc_ref[...] += jnp.sum(beats, axis=1, keepdims=True)

    @pl.when(j == NB - 1)
    def _last():
        thr2 = jnp.where(iota < (N - j * BLK), thr, jnp.inf)
        beats = (x > thr2).astype(jnp.float32)
        counts = acc_ref[...] + jnp.sum(beats, axis=1, keepdims=True)
        hits = (counts < K).astype(jnp.float32)
        out_ref[...] = (jnp.sum(hits) * (1.0 / B)).reshape(1, 1)


def kernel(logits, target):
    tgt = target.astype(jnp.int32)

    grid_spec = pltpu.PrefetchScalarGridSpec(
        num_scalar_prefetch=1,
        grid=(B,),
        in_specs=[pl.BlockSpec((8, GBLK), lambda b, t: (b // 8, t[b] // GBLK))],
        out_specs=pl.BlockSpec((B, 1), lambda b, t: (0, 0)),
    )
    tv = pl.pallas_call(
        _gather_body,
        grid_spec=grid_spec,
        out_shape=jax.ShapeDtypeStruct((B, 1), jnp.float32),
    )(tgt, logits)

    flag = pl.pallas_call(
        _count_fast_body,
        grid=(NB,),
        in_specs=[
            pl.BlockSpec((B, 1), lambda j: (0, 0)),
            pl.BlockSpec((B, BLK), lambda j: (0, j)),
        ],
        out_specs=pl.BlockSpec((1, 1), lambda j: (0, 0)),
        out_shape=jax.ShapeDtypeStruct((1, 1), jnp.float32),
        scratch_shapes=[pltpu.VMEM((B, 1), jnp.float32)],
    )(tv, logits)

    def _exact(_):
        tvm = jnp.nextafter(tv, jnp.float32(-jnp.inf))
        acc = pl.pallas_call(
            _count_exact_body,
            grid=(NB,),
            in_specs=[
                pl.BlockSpec((B, 1), lambda j: (0, 0)),
                pl.BlockSpec((B, 1), lambda j: (0, 0)),
                pl.BlockSpec((B, 1), lambda j: (0, 0)),
                pl.BlockSpec((B, BLK), lambda j: (0, j)),
            ],
            out_specs=pl.BlockSpec((1, 1), lambda j: (0, 0)),
            out_shape=jax.ShapeDtypeStruct((1, 1), jnp.float32),
            scratch_shapes=[pltpu.VMEM((B, 1), jnp.float32)],
        )(tv, tvm, tgt.reshape(B, 1), logits)
        return acc[0, 0]

    return lax.cond(flag[0, 0] > 0, _exact, lambda _: jnp.float32(0.0), None)
